# baseline (speedup 1.0000x reference)
import jax
import jax.numpy as jnp
from jax.experimental import pallas as pl
from jax.experimental.pallas import tpu as pltpu

_H = 64
_W = 64
_WP = _W + 2
_NPIX = _H * _WP
_XCOLS = 4480
_TILE = 1408
_NT = _NPIX // _TILE


def _body(x_ref, w1_ref, beta_ref, wc2_ref, wr2_ref, bc2_ref, br2_ref,
          cls_ref, reg_ref):
    j0 = pl.program_id(0) * _TILE
    xw = x_ref[:, pl.ds(j0, _TILE + 256)]
    acc = jnp.zeros((512, _TILE), jnp.float32)
    for k in range(9):
        dy, dx = divmod(k, 3)
        off = dy * _WP + dx
        xs = jax.lax.slice(xw, (0, off), (256, off + _TILE))
        acc = acc + jax.lax.dot_general(
            w1_ref[k], xs, (((1,), (0,)), ((), ())),
            preferred_element_type=jnp.float32)
    acc = acc + beta_ref[:]
    h = (acc * jax.nn.sigmoid(acc)).astype(jnp.bfloat16)
    cls_ref[:, :] = jax.lax.dot_general(
        wc2_ref[:], h[:256], (((1,), (0,)), ((), ())),
        preferred_element_type=jnp.float32) + bc2_ref[:]
    reg_ref[:, :] = jax.lax.dot_general(
        wr2_ref[:], h[256:], (((1,), (0,)), ((), ())),
        preferred_element_type=jnp.float32) + br2_ref[:]


def kernel(feats, strides, training, cls_w1, cls_gamma, cls_beta, cls_w2,
           cls_b2, reg_w1, reg_gamma, reg_beta, reg_w2, reg_b2):
    x = feats[0, 0]
    xp = jnp.pad(x, ((0, 0), (1, 1), (1, 1)))
    xflat = jnp.pad(xp.reshape(256, _WP * (_H + 2)),
                    ((0, 0), (0, _XCOLS - _WP * (_H + 2)))).astype(jnp.bfloat16)

    eps = 1e-5
    sc = cls_gamma / jnp.sqrt(1.0 + eps)
    sr = reg_gamma / jnp.sqrt(1.0 + eps)
    w1 = jnp.concatenate([cls_w1 * sc[:, None, None, None],
                          reg_w1 * sr[:, None, None, None]], axis=0)
    w1 = w1.transpose(2, 3, 0, 1).reshape(9, 512, 256).astype(jnp.bfloat16)
    beta = jnp.concatenate([cls_beta, reg_beta])[:, None]
    wc2 = cls_w2[:, :, 0, 0].astype(jnp.bfloat16)
    wr2 = jnp.pad(reg_w2[:, :, 0, 0], ((0, 4), (0, 0))).astype(jnp.bfloat16)
    bc2 = cls_b2[:, None]
    br2 = jnp.pad(reg_b2, (0, 4))[:, None]

    cls_flat, reg_flat = pl.pallas_call(
        _body,
        grid=(_NT,),
        in_specs=[
            pl.BlockSpec((256, _XCOLS), lambda i: (0, 0)),
            pl.BlockSpec((9, 512, 256), lambda i: (0, 0, 0)),
            pl.BlockSpec((512, 1), lambda i: (0, 0)),
            pl.BlockSpec((80, 256), lambda i: (0, 0)),
            pl.BlockSpec((72, 256), lambda i: (0, 0)),
            pl.BlockSpec((80, 1), lambda i: (0, 0)),
            pl.BlockSpec((72, 1), lambda i: (0, 0)),
        ],
        out_specs=[
            pl.BlockSpec((80, _TILE), lambda i: (0, i)),
            pl.BlockSpec((72, _TILE), lambda i: (0, i)),
        ],
        out_shape=[
            jax.ShapeDtypeStruct((80, _NPIX), jnp.float32),
            jax.ShapeDtypeStruct((72, _NPIX), jnp.float32),
        ],
        compiler_params=pltpu.CompilerParams(
            dimension_semantics=("arbitrary",)),
    )(xflat, w1, beta, wc2, wr2, bc2, br2)

    cls = cls_flat.reshape(80, _H, _WP)[:, :, :_W][None]
    reg = reg_flat[:68].reshape(68, _H, _WP)[:, :, :_W][None]
    return (cls, reg)


# speedup vs baseline: 1.0248x; 1.0248x over previous
import jax
import jax.numpy as jnp
from jax.experimental import pallas as pl
from jax.experimental.pallas import tpu as pltpu

_H = 64
_W = 64
_WP = _W + 2
_NPIX = _H * _WP
_XCOLS = 4480
_TILE = 1408
_NT = _NPIX // _TILE


def _body(x_ref, w1_ref, scale_ref, beta_ref, wc2_ref, wr2_ref, bc2_ref,
          br2_ref, cls_ref, reg_ref):
    j0 = pl.program_id(0) * _TILE
    xw = x_ref[:, pl.ds(j0, _TILE + 256)]
    acc = jnp.zeros((512, _TILE), jnp.float32)
    for k in range(9):
        dy, dx = divmod(k, 3)
        off = dy * _WP + dx
        xs = jax.lax.slice(xw, (0, off), (256, off + _TILE))
        acc = acc + jax.lax.dot_general(
            w1_ref[k], xs, (((1,), (0,)), ((), ())),
            preferred_element_type=jnp.float32)
    acc = acc * scale_ref[:] + beta_ref[:]
    h = (acc * jax.nn.sigmoid(acc)).astype(jnp.bfloat16)
    cls_ref[:, :] = jax.lax.dot_general(
        wc2_ref[:], h[:256], (((1,), (0,)), ((), ())),
        preferred_element_type=jnp.float32) + bc2_ref[:]
    reg_ref[:, :] = jax.lax.dot_general(
        wr2_ref[:], h[256:], (((1,), (0,)), ((), ())),
        preferred_element_type=jnp.float32) + br2_ref[:]


def kernel(feats, strides, training, cls_w1, cls_gamma, cls_beta, cls_w2,
           cls_b2, reg_w1, reg_gamma, reg_beta, reg_w2, reg_b2):
    x = feats[0, 0]
    xp = jnp.pad(x, ((0, 0), (1, 1), (1, 1)))
    xflat = jnp.pad(xp.reshape(256, _WP * (_H + 2)),
                    ((0, 0), (0, _XCOLS - _WP * (_H + 2)))).astype(jnp.bfloat16)

    eps = 1e-5
    scale = (jnp.concatenate([cls_gamma, reg_gamma])
             / jnp.sqrt(1.0 + eps))[:, None]
    w1 = jnp.concatenate([cls_w1, reg_w1], axis=0).astype(jnp.bfloat16)
    w1 = w1.transpose(2, 3, 0, 1).reshape(9, 512, 256)
    beta = jnp.concatenate([cls_beta, reg_beta])[:, None]
    wc2 = cls_w2[:, :, 0, 0].astype(jnp.bfloat16)
    wr2 = jnp.pad(reg_w2[:, :, 0, 0], ((0, 4), (0, 0))).astype(jnp.bfloat16)
    bc2 = cls_b2[:, None]
    br2 = jnp.pad(reg_b2, (0, 4))[:, None]

    cls_flat, reg_flat = pl.pallas_call(
        _body,
        grid=(_NT,),
        in_specs=[
            pl.BlockSpec((256, _XCOLS), lambda i: (0, 0)),
            pl.BlockSpec((9, 512, 256), lambda i: (0, 0, 0)),
            pl.BlockSpec((512, 1), lambda i: (0, 0)),
            pl.BlockSpec((512, 1), lambda i: (0, 0)),
            pl.BlockSpec((80, 256), lambda i: (0, 0)),
            pl.BlockSpec((72, 256), lambda i: (0, 0)),
            pl.BlockSpec((80, 1), lambda i: (0, 0)),
            pl.BlockSpec((72, 1), lambda i: (0, 0)),
        ],
        out_specs=[
            pl.BlockSpec((80, _TILE), lambda i: (0, i)),
            pl.BlockSpec((72, _TILE), lambda i: (0, i)),
        ],
        out_shape=[
            jax.ShapeDtypeStruct((80, _NPIX), jnp.float32),
            jax.ShapeDtypeStruct((72, _NPIX), jnp.float32),
        ],
        compiler_params=pltpu.CompilerParams(
            dimension_semantics=("arbitrary",)),
    )(xflat, w1, scale, beta, wc2, wr2, bc2, br2)

    cls = cls_flat.reshape(80, _H, _WP)[:, :, :_W][None]
    reg = reg_flat[:68].reshape(68, _H, _WP)[:, :, :_W][None]
    return (cls, reg)


# speedup vs baseline: 1.5675x; 1.5296x over previous
import jax
import jax.numpy as jnp
from jax.experimental import pallas as pl
from jax.experimental.pallas import tpu as pltpu

_N = 64 * 64
_PAD = 128
_XC = _N + 2 * _PAD
_TILE = 2048
_NT = _N // _TILE
_RSQ = 0.9999950000374997


def _body(x_ref, w1_ref, gc_ref, bc_ref, gr_ref, br_ref,
          wc2_ref, bc2_ref, wr2_ref, br2_ref, cls_ref, reg_ref,
          xpad, svec, bvec):
    i = pl.program_id(0)

    @pl.when(i == 0)
    def _init():
        xpad[:, :_PAD] = jnp.zeros((256, _PAD), jnp.bfloat16)
        xpad[:, _N + _PAD:] = jnp.zeros((256, _PAD), jnp.bfloat16)
        xpad[:, _PAD:_N + _PAD] = x_ref[:, :].astype(jnp.bfloat16)
        svec[:256] = gc_ref[0].reshape(256, 1) * _RSQ
        svec[256:] = gr_ref[0].reshape(256, 1) * _RSQ
        bvec[:256] = bc_ref[0].reshape(256, 1)
        bvec[256:] = br_ref[0].reshape(256, 1)

    j0 = i * _TILE
    xw = xpad[:, pl.ds(j0, _TILE + 2 * _PAD)]
    lane = jax.lax.broadcasted_iota(jnp.int32, (1, _TILE), 1)
    m0 = (lane % 64 != 0).astype(jnp.bfloat16)
    m2 = (lane % 64 != 63).astype(jnp.bfloat16)
    acc = jnp.zeros((512, _TILE), jnp.float32)
    for k in range(9):
        dy, dx = divmod(k, 3)
        off = _PAD + (dy - 1) * 64 + (dx - 1)
        xs = jax.lax.slice(xw, (0, off), (256, off + _TILE))
        if dx == 0:
            xs = xs * m0
        elif dx == 2:
            xs = xs * m2
        acc = acc + jax.lax.dot_general(
            w1_ref[k], xs, (((1,), (0,)), ((), ())),
            preferred_element_type=jnp.float32)
    acc = acc * svec[:, :1] + bvec[:, :1]
    h = (acc * jax.nn.sigmoid(acc)).astype(jnp.bfloat16)
    cls_ref[:, :] = jax.lax.dot_general(
        wc2_ref[:, :].astype(jnp.bfloat16), h[:256], (((1,), (0,)), ((), ())),
        preferred_element_type=jnp.float32) + bc2_ref[0].reshape(80, 1)
    reg_ref[:, :] = jax.lax.dot_general(
        wr2_ref[:, :].astype(jnp.bfloat16), h[256:], (((1,), (0,)), ((), ())),
        preferred_element_type=jnp.float32) + br2_ref[0].reshape(68, 1)


def kernel(feats, strides, training, cls_w1, cls_gamma, cls_beta, cls_w2,
           cls_b2, reg_w1, reg_gamma, reg_beta, reg_w2, reg_b2):
    w1 = jnp.concatenate([cls_w1, reg_w1], axis=0).astype(jnp.bfloat16)
    w1 = w1.reshape(512, 256, 9).transpose(2, 0, 1)
    full = lambda *dims: pl.BlockSpec(dims, lambda i: tuple(0 for _ in dims))
    cls_flat, reg_flat = pl.pallas_call(
        _body,
        grid=(_NT,),
        in_specs=[
            full(256, _N),
            full(9, 512, 256),
            full(1, 256), full(1, 256), full(1, 256), full(1, 256),
            full(80, 256), full(1, 80), full(68, 256), full(1, 68),
        ],
        out_specs=[
            pl.BlockSpec((80, _TILE), lambda i: (0, i)),
            pl.BlockSpec((68, _TILE), lambda i: (0, i)),
        ],
        out_shape=[
            jax.ShapeDtypeStruct((80, _N), jnp.float32),
            jax.ShapeDtypeStruct((68, _N), jnp.float32),
        ],
        scratch_shapes=[
            pltpu.VMEM((256, _XC), jnp.bfloat16),
            pltpu.VMEM((512, 1), jnp.float32),
            pltpu.VMEM((512, 1), jnp.float32),
        ],
        compiler_params=pltpu.CompilerParams(
            dimension_semantics=("arbitrary",)),
    )(feats.reshape(256, _N), w1, cls_gamma.reshape(1, 256),
      cls_beta.reshape(1, 256), reg_gamma.reshape(1, 256),
      reg_beta.reshape(1, 256), cls_w2.reshape(80, 256),
      cls_b2.reshape(1, 80), reg_w2.reshape(68, 256),
      reg_b2.reshape(1, 68))
    return (cls_flat.reshape(1, 80, 64, 64), reg_flat.reshape(1, 68, 64, 64))
